# baseline (device time: 58069 ns/iter reference)
import jax
import jax.numpy as jnp
from jax import lax
from jax.experimental import pallas as pl
from jax.experimental.pallas import tpu as pltpu

N_DEV = 16
N_STEPS = 4
N_LAYERS = 3
N_EXCH = N_LAYERS * N_STEPS
MASKS = [1, 3, 4, 8]


def kernel(x, Win0, Wout0, Win1, Wout1, Win2, Wout2):
    b, d_loc = x.shape
    _, h_dim = Win0.shape

    def body(x_ref, win0_ref, wout0_ref, win1_ref, wout1_ref, win2_ref,
             wout2_ref, out_ref, send_ref, recv_ref, win_scr, wout_scr,
             send_sems, recv_sems, win_copy_sems, wout_copy_sems):
        my = lax.axis_index("i")
        partners = [my ^ m for m in MASKS]

        barrier_sem = pltpu.get_barrier_semaphore()
        for p in partners:
            pl.semaphore_signal(barrier_sem, inc=1, device_id=(p,),
                                device_id_type=pl.DeviceIdType.MESH)

        win_hbm = [win0_ref, win1_ref, win2_ref]
        wout_hbm = [wout0_ref, wout1_ref, wout2_ref]
        win_copies = []
        wout_copies = []
        for l in range(N_LAYERS):
            cw = pltpu.make_async_copy(win_hbm[l], win_scr.at[l],
                                       win_copy_sems.at[l])
            cw.start()
            co = pltpu.make_async_copy(wout_hbm[l], wout_scr.at[l],
                                       wout_copy_sems.at[l])
            co.start()
            win_copies.append(cw)
            wout_copies.append(co)

        xb = x_ref[...].astype(jnp.bfloat16)
        for layer in range(N_LAYERS):
            win_copies[layer].wait()
            send_ref[layer * N_STEPS] = jnp.dot(
                xb, win_scr[layer].astype(jnp.bfloat16),
                preferred_element_type=jnp.float32).astype(jnp.bfloat16)
            if layer == 0:
                pl.semaphore_wait(barrier_sem, N_STEPS)

            h = None
            wout_b = None
            for k in range(N_STEPS):
                ex = layer * N_STEPS + k
                rdma = pltpu.make_async_remote_copy(
                    src_ref=send_ref.at[ex],
                    dst_ref=recv_ref.at[ex],
                    send_sem=send_sems.at[ex],
                    recv_sem=recv_sems.at[ex],
                    device_id=(partners[k],),
                    device_id_type=pl.DeviceIdType.MESH,
                )
                rdma.start()
                if k == 0:
                    wout_copies[layer].wait()
                    wout_b = wout_scr[layer].astype(jnp.bfloat16)
                rdma.wait_recv()
                if k < N_STEPS - 1:
                    send_ref[ex + 1] = send_ref[ex] + recv_ref[ex]
                else:
                    h = jnp.maximum(send_ref[ex] + recv_ref[ex], 0)

            if layer == N_LAYERS - 1:
                out_ref[...] = jnp.dot(h, wout_b,
                                       preferred_element_type=jnp.float32)
            else:
                xb = jnp.dot(h, wout_b,
                             preferred_element_type=jnp.float32
                             ).astype(jnp.bfloat16)

        for ex in range(N_EXCH):
            drain = pltpu.make_async_remote_copy(
                src_ref=send_ref.at[ex],
                dst_ref=recv_ref.at[ex],
                send_sem=send_sems.at[ex],
                recv_sem=recv_sems.at[ex],
                device_id=(my,),
                device_id_type=pl.DeviceIdType.MESH,
            )
            drain.wait_send()

    return pl.pallas_call(
        body,
        out_shape=jax.ShapeDtypeStruct((b, d_loc), jnp.float32),
        in_specs=[pl.BlockSpec(memory_space=pltpu.VMEM)]
        + [pl.BlockSpec(memory_space=pltpu.MemorySpace.HBM)] * 6,
        out_specs=pl.BlockSpec(memory_space=pltpu.VMEM),
        scratch_shapes=[
            pltpu.VMEM((N_EXCH, b, h_dim), jnp.bfloat16),
            pltpu.VMEM((N_EXCH, b, h_dim), jnp.bfloat16),
            pltpu.VMEM((N_LAYERS, d_loc, h_dim), jnp.float32),
            pltpu.VMEM((N_LAYERS, h_dim, d_loc), jnp.float32),
            pltpu.SemaphoreType.DMA((N_EXCH,)),
            pltpu.SemaphoreType.DMA((N_EXCH,)),
            pltpu.SemaphoreType.DMA((N_LAYERS,)),
            pltpu.SemaphoreType.DMA((N_LAYERS,)),
        ],
        compiler_params=pltpu.CompilerParams(collective_id=0),
    )(x, Win0, Wout0, Win1, Wout1, Win2, Wout2)


# device time: 53198 ns/iter; 1.0916x vs baseline; 1.0916x over previous
import jax
import jax.numpy as jnp
from jax import lax
from jax.experimental import pallas as pl
from jax.experimental.pallas import tpu as pltpu

N_DEV = 16
N_STEPS = 4
N_LAYERS = 3
N_EXCH = N_LAYERS * N_STEPS
MASKS = [1, 3, 4, 8]


def kernel(x, Win0, Wout0, Win1, Wout1, Win2, Wout2):
    b, d_loc = x.shape
    _, h_dim = Win0.shape
    hh = h_dim // 2

    def body(x_ref, win0_ref, wout0_ref, win1_ref, wout1_ref, win2_ref,
             wout2_ref, out_ref, s1_ref, r1_ref, s2_ref, r2_ref,
             win_scr, wout_scr, send1_sems, recv1_sems, send2_sems,
             recv2_sems, win_copy_sems, wout_copy_sems):
        my = lax.axis_index("i")
        partners = [my ^ m for m in MASKS]

        barrier_sem = pltpu.get_barrier_semaphore()
        for p in partners:
            pl.semaphore_signal(barrier_sem, inc=1, device_id=(p,),
                                device_id_type=pl.DeviceIdType.MESH)

        win_hbm = [win0_ref, win1_ref, win2_ref]
        wout_hbm = [wout0_ref, wout1_ref, wout2_ref]
        win_copies = []
        wout_copies = []
        for l in range(N_LAYERS):
            cw = pltpu.make_async_copy(win_hbm[l], win_scr.at[l],
                                       win_copy_sems.at[l])
            cw.start()
            co = pltpu.make_async_copy(wout_hbm[l], wout_scr.at[l],
                                       wout_copy_sems.at[l])
            co.start()
            win_copies.append(cw)
            wout_copies.append(co)

        def exchange(chain, ex, k):
            s_ref, r_ref, ssems, rsems = (
                (s1_ref, r1_ref, send1_sems, recv1_sems) if chain == 1
                else (s2_ref, r2_ref, send2_sems, recv2_sems))
            return pltpu.make_async_remote_copy(
                src_ref=s_ref.at[ex],
                dst_ref=r_ref.at[ex],
                send_sem=ssems.at[ex],
                recv_sem=rsems.at[ex],
                device_id=(partners[k],),
                device_id_type=pl.DeviceIdType.MESH,
            )

        xb = x_ref[...].astype(jnp.bfloat16)
        for layer in range(N_LAYERS):
            ex0 = layer * N_STEPS
            win_copies[layer].wait()
            win_b = win_scr[layer].astype(jnp.bfloat16)
            s1_ref[ex0] = jnp.dot(
                xb, win_b[:, :hh],
                preferred_element_type=jnp.float32).astype(jnp.bfloat16)
            if layer == 0:
                pl.semaphore_wait(barrier_sem, N_STEPS)
            rd1 = exchange(1, ex0, 0)
            rd1.start()
            s2_ref[ex0] = jnp.dot(
                xb, win_b[:, hh:],
                preferred_element_type=jnp.float32).astype(jnp.bfloat16)
            rd2 = exchange(2, ex0, 0)
            rd2.start()

            wout_b = None
            for k in range(N_STEPS - 1):
                ex = ex0 + k
                if k == 0:
                    wout_copies[layer].wait()
                    wout_b = wout_scr[layer].astype(jnp.bfloat16)
                rd1.wait_recv()
                s1_ref[ex + 1] = s1_ref[ex] + r1_ref[ex]
                nxt1 = exchange(1, ex + 1, k + 1)
                nxt1.start()
                rd2.wait_recv()
                s2_ref[ex + 1] = s2_ref[ex] + r2_ref[ex]
                nxt2 = exchange(2, ex + 1, k + 1)
                nxt2.start()
                rd1, rd2 = nxt1, nxt2

            exl = ex0 + N_STEPS - 1
            rd1.wait_recv()
            h1 = jnp.maximum(s1_ref[exl] + r1_ref[exl], 0)
            acc = jnp.dot(h1, wout_b[:hh, :],
                          preferred_element_type=jnp.float32)
            rd2.wait_recv()
            h2 = jnp.maximum(s2_ref[exl] + r2_ref[exl], 0)
            xnext = acc + jnp.dot(h2, wout_b[hh:, :],
                                  preferred_element_type=jnp.float32)
            if layer == N_LAYERS - 1:
                out_ref[...] = xnext
            else:
                xb = xnext.astype(jnp.bfloat16)

        for ex in range(N_EXCH):
            for chain in (1, 2):
                exchange(chain, ex, 0).wait_send()

    return pl.pallas_call(
        body,
        out_shape=jax.ShapeDtypeStruct((b, d_loc), jnp.float32),
        in_specs=[pl.BlockSpec(memory_space=pltpu.VMEM)]
        + [pl.BlockSpec(memory_space=pltpu.MemorySpace.HBM)] * 6,
        out_specs=pl.BlockSpec(memory_space=pltpu.VMEM),
        scratch_shapes=[
            pltpu.VMEM((N_EXCH, b, hh), jnp.bfloat16),
            pltpu.VMEM((N_EXCH, b, hh), jnp.bfloat16),
            pltpu.VMEM((N_EXCH, b, hh), jnp.bfloat16),
            pltpu.VMEM((N_EXCH, b, hh), jnp.bfloat16),
            pltpu.VMEM((N_LAYERS, d_loc, h_dim), jnp.float32),
            pltpu.VMEM((N_LAYERS, h_dim, d_loc), jnp.float32),
            pltpu.SemaphoreType.DMA((N_EXCH,)),
            pltpu.SemaphoreType.DMA((N_EXCH,)),
            pltpu.SemaphoreType.DMA((N_EXCH,)),
            pltpu.SemaphoreType.DMA((N_EXCH,)),
            pltpu.SemaphoreType.DMA((N_LAYERS,)),
            pltpu.SemaphoreType.DMA((N_LAYERS,)),
        ],
        compiler_params=pltpu.CompilerParams(collective_id=0),
    )(x, Win0, Wout0, Win1, Wout1, Win2, Wout2)
